# baseline (speedup 1.0000x reference)
import functools

import jax
import jax.numpy as jnp
from jax import lax
from jax.experimental import pallas as pl
from jax.experimental.pallas import tpu as pltpu
from jax.experimental.pallas import tpu_sc as plsc

_N = 50000
_D = 256
_B = 1024
_K = 128
_L = 32

_NC = 2
_NS = 16
_NW = _NC * _NS
_GPW = _B // _NW
_LANES = 16
_CHUNKS = _D // _LANES
_LN2 = 0.6931471805599453
_SQRT2 = 1.4142135623730951


def _sc_pool_head(table, idx, W, b):
    mesh = plsc.VectorSubcoreMesh(core_axis_name="c", subcore_axis_name="s")

    @functools.partial(
        pl.kernel,
        mesh=mesh,
        out_type=jax.ShapeDtypeStruct((_B * _L,), jnp.float32),
        scratch_types=[
            pltpu.VMEM((_GPW, _K), jnp.int32),
            pltpu.VMEM((3, _K, 128), jnp.float32),
            pltpu.VMEM((_GPW, _D), jnp.float32),
            pltpu.VMEM((_D * _L,), jnp.float32),
            pltpu.VMEM((_L,), jnp.float32),
            pltpu.VMEM((_GPW * _L,), jnp.float32),
            pltpu.SemaphoreType.DMA,
            pltpu.SemaphoreType.DMA,
            pltpu.SemaphoreType.DMA,
        ],
    )
    def sc_kernel(table_hbm, idx_hbm, w_hbm, b_hbm, out_hbm,
                  idx_v, rows_v, pooled_v, w_v, b_v, out_v,
                  sem0, sem1, sem2):
        sems = (sem0, sem1, sem2)
        wid = lax.axis_index("s") * _NC + lax.axis_index("c")
        base = wid * _GPW
        pltpu.sync_copy(idx_hbm.at[pl.ds(base, _GPW)], idx_v)
        pltpu.sync_copy(w_hbm, w_v)
        pltpu.sync_copy(b_hbm, b_v)

        def gather(j, slot):
            pltpu.async_copy(table_hbm.at[idx_v.at[j]], rows_v.at[slot],
                             sems[slot])

        def accumulate(j, slot):
            pltpu.make_async_copy(table_hbm.at[idx_v.at[j]],
                                  rows_v.at[slot], sems[slot]).wait()
            buf = rows_v.at[slot]

            def body(r, accs):
                r2 = 2 * r
                return tuple(
                    accs[c]
                    + buf[r2, pl.ds((c % 8) * _LANES, _LANES)]
                    + buf[r2 + 1, pl.ds((c % 8) * _LANES, _LANES)]
                    for c in range(_CHUNKS)
                )

            zeros = tuple(
                jnp.zeros((_LANES,), jnp.float32) for _ in range(_CHUNKS)
            )
            accs = lax.fori_loop(0, _K // 2, body, zeros)
            for c in range(_CHUNKS):
                pooled_v[j, pl.ds(c * _LANES, _LANES)] = accs[c]

        gdn = lax.GatherDimensionNumbers(
            offset_dims=(), collapsed_slice_dims=(0,), start_index_map=(0,))

        def _perm(x, idxv):
            return lax.gather(
                x, idxv.reshape(_LANES, 1), gdn, (1,),
                mode=lax.GatherScatterMode.PROMISE_IN_BOUNDS)

        iot = lax.iota(jnp.int32, _LANES)

        def head(g):
            def mv_body(c, carry):
                l0, l1 = carry
                chunk = pooled_v[g, pl.ds(c * _LANES, _LANES)]
                for jj in range(_LANES):
                    d = c * _LANES + jj
                    s = _perm(chunk, jnp.full((_LANES,), jj, jnp.int32))
                    l0 = l0 + s * w_v[pl.ds(d * _L, _LANES)]
                    l1 = l1 + s * w_v[pl.ds(d * _L + _LANES, _LANES)]
                return (l0, l1)

            l0, l1 = lax.fori_loop(
                0, _CHUNKS, mv_body,
                (b_v[pl.ds(0, _LANES)], b_v[pl.ds(_LANES, _LANES)]))

            mxv = jnp.maximum(l0, l1)
            for k in (1, 2, 4, 8):
                mxv = jnp.maximum(mxv, _perm(mxv, iot ^ k))
            s0 = l0 - mxv
            s1 = l1 - mxv
            sv = jnp.exp(s0) + jnp.exp(s1)
            for k in (1, 2, 4, 8):
                sv = sv + _perm(sv, iot ^ k)
            ib = lax.bitcast_convert_type(sv, jnp.int32)
            ex = lax.shift_right_arithmetic(ib, 23) - 127
            mant = lax.bitcast_convert_type(
                lax.bitwise_or(lax.bitwise_and(ib, 0x007FFFFF), 0x3F800000),
                jnp.float32)
            big = mant > _SQRT2
            mant = jnp.where(big, mant * 0.5, mant)
            exf = jnp.where(big, ex + 1, ex).astype(jnp.float32)
            z = (mant - 1.0) / (mant + 1.0)
            z2 = z * z
            logm = 2.0 * z * (1.0 + z2 * (1.0 / 3.0 + z2 * (0.2 + z2 / 7.0)))
            logs = exf * _LN2 + logm
            out_v[pl.ds(g * _L, _LANES)] = s0 - logs
            out_v[pl.ds(g * _L + _LANES, _LANES)] = s1 - logs

        gather(0, 0)
        gather(1, 1)
        gather(2, 2)

        def outer(t, _):
            j0 = 3 * t
            for u in range(3):
                accumulate(j0 + u, u)
                gather(j0 + u + 3, u)
                head(j0 + u)
            return 0

        lax.fori_loop(0, _GPW // 3 - 1, outer, 0)
        accumulate(27, 0)
        gather(30, 0)
        head(27)
        accumulate(28, 1)
        gather(31, 1)
        head(28)
        accumulate(29, 2)
        head(29)
        accumulate(30, 0)
        head(30)
        accumulate(31, 1)
        head(31)

        pltpu.sync_copy(out_v, out_hbm.at[pl.ds(base * _L, _GPW * _L)])

    return sc_kernel(table.reshape(-1, 128), idx * 2, W.reshape(-1), b).reshape(_B, _L)


def kernel(node_embedding_matrix, batch_x_index, W, b):
    return _sc_pool_head(node_embedding_matrix, batch_x_index, W, b)


# speedup vs baseline: 1.4791x; 1.4791x over previous
import functools

import jax
import jax.numpy as jnp
from jax import lax
from jax.experimental import pallas as pl
from jax.experimental.pallas import tpu as pltpu
from jax.experimental.pallas import tpu_sc as plsc

_N = 50000
_D = 256
_B = 1024
_K = 128
_L = 32

_NC = 2
_NS = 16
_NW = _NC * _NS
_GPW = _B // _NW
_LANES = 16
_CHUNKS = _D // _LANES


def _pooled_sparsecore(table, idx_flat):
    mesh = plsc.VectorSubcoreMesh(core_axis_name="c", subcore_axis_name="s")

    @functools.partial(
        pl.kernel,
        mesh=mesh,
        out_type=jax.ShapeDtypeStruct((_B, _D), jnp.float32),
        scratch_types=[
            pltpu.VMEM((_GPW * _K,), jnp.int32),
            pltpu.VMEM((3, _K, _D), jnp.float32),
            pltpu.VMEM((_GPW, _D), jnp.float32),
            pltpu.SemaphoreType.DMA,
            pltpu.SemaphoreType.DMA,
            pltpu.SemaphoreType.DMA,
        ],
    )
    def sc_kernel(table_hbm, idx_hbm, out_hbm, idx_v, rows_v, pooled_v,
                  sem0, sem1, sem2):
        sems = (sem0, sem1, sem2)
        wid = lax.axis_index("s") * _NC + lax.axis_index("c")
        base = wid * _GPW
        pltpu.sync_copy(idx_hbm.at[pl.ds(base * _K, _GPW * _K)], idx_v)

        def gather(j, slot):
            pltpu.async_copy(table_hbm.at[idx_v.at[pl.ds(j * _K, _K)]],
                             rows_v.at[slot], sems[slot])

        def accumulate(j, slot):
            pltpu.make_async_copy(table_hbm.at[idx_v.at[pl.ds(j * _K, _K)]],
                                  rows_v.at[slot], sems[slot]).wait()
            buf = rows_v.at[slot]

            def body(r, accs):
                r2 = 2 * r
                return tuple(
                    accs[c]
                    + buf[r2, pl.ds(c * _LANES, _LANES)]
                    + buf[r2 + 1, pl.ds(c * _LANES, _LANES)]
                    for c in range(_CHUNKS)
                )

            zeros = tuple(
                jnp.zeros((_LANES,), jnp.float32) for _ in range(_CHUNKS)
            )
            accs = lax.fori_loop(0, _K // 2, body, zeros)
            for c in range(_CHUNKS):
                pooled_v[j, pl.ds(c * _LANES, _LANES)] = accs[c]

        gather(0, 0)
        gather(1, 1)
        gather(2, 2)

        def outer(t, _):
            j0 = 3 * t
            for u in range(3):
                accumulate(j0 + u, u)
                gather(j0 + u + 3, u)
            return 0

        lax.fori_loop(0, _GPW // 3 - 1, outer, 0)
        accumulate(27, 0)
        gather(30, 0)
        accumulate(28, 1)
        gather(31, 1)
        accumulate(29, 2)
        accumulate(30, 0)
        accumulate(31, 1)

        pltpu.sync_copy(pooled_v, out_hbm.at[pl.ds(base, _GPW)])

    return sc_kernel(table, idx_flat)


def _head_kernel(pooled_ref, w_ref, b_ref, out_ref):
    logits = (
        jnp.dot(pooled_ref[...], w_ref[...],
                preferred_element_type=jnp.float32)
        + b_ref[...][None, :]
    )
    m = jnp.max(logits, axis=1, keepdims=True)
    shifted = logits - m
    lse = jnp.log(jnp.sum(jnp.exp(shifted), axis=1, keepdims=True))
    out_ref[...] = shifted - lse


def _head(pooled, W, b):
    return pl.pallas_call(
        _head_kernel,
        out_shape=jax.ShapeDtypeStruct((_B, _L), jnp.float32),
    )(pooled, W, b)


def kernel(node_embedding_matrix, batch_x_index, W, b):
    pooled = _pooled_sparsecore(node_embedding_matrix,
                                batch_x_index.reshape(-1))
    return _head(pooled, W, b)
